# baseline (speedup 1.0000x reference)
import functools

import jax
import jax.numpy as jnp
from jax import lax
from jax.experimental import pallas as pl
from jax.experimental.pallas import tpu as pltpu
from jax.experimental.pallas import tpu_sc as plsc

N = 10000
E = 320000
D = 128
G = 32
C = 10
NACC = 10112
NWORKERS = 32
EPT = E // NWORKERS
CH = 80
NCHUNK = EPT // CH
DEPTH = 4
SUP = 5
NSUP = NCHUNK // SUP
ROWS_PER_TILE = NACC // 16


def _sc_agg_body(src_hbm, dst_hbm, table_hbm, zeros_hbm, out_hbm,
                 isrcA, idstA, isrcB, idstB, rows0, rows1, rows2, rows3, acc,
                 sem0, sem1, sem2, sem3, sc0, sc1, sc2, sc3,
                 ssemA, ssemB, dsemA, dsemB, zsem):
    cid = lax.axis_index("c")
    sid = lax.axis_index("s")
    wid = cid * 16 + sid
    rows = [rows0, rows1, rows2, rows3]
    sems = [sem0, sem1, sem2, sem3]
    scs = [sc0, sc1, sc2, sc3]
    isrc = [isrcA, isrcB]
    idst = [idstA, idstB]
    ssem = [ssemA, ssemB]
    dsem = [dsemA, dsemB]

    zdst = acc.at[pl.ds(sid * ROWS_PER_TILE, ROWS_PER_TILE)]
    pltpu.async_copy(zeros_hbm, zdst, zsem)
    pltpu.sync_copy(src_hbm.at[wid, 0], isrcA)
    pltpu.sync_copy(dst_hbm.at[wid, 0], idstA)
    pltpu.async_copy(src_hbm.at[wid, 1], isrcB, ssemB)
    pltpu.async_copy(dst_hbm.at[wid, 1], idstB, dsemB)
    pltpu.make_async_copy(zeros_hbm, zdst, zsem).wait()
    plsc.subcore_barrier()

    def gstart(c, k):
        p, j = (c // SUP) % 2, c % SUP
        pltpu.async_copy(table_hbm.at[isrc[p].at[j]], rows[k], sems[k])

    def gwait(k):
        pltpu.make_async_copy(table_hbm.at[isrcA.at[0]], rows[k], sems[k]).wait()

    def scat(c, k):
        p, j = (c // SUP) % 2, c % SUP
        pltpu.async_copy(rows[k], acc.at[idst[p].at[j]], scs[k], add=True)

    def scwait(k):
        pltpu.make_async_copy(rows[k], acc.at[idstA.at[0]], scs[k]).wait()

    for c in range(DEPTH - 1):
        gstart(c, c)
    for c in range(NCHUNK):
        k = c % DEPTH
        s = c // SUP
        if c % SUP == 0 and s >= 1:
            pltpu.make_async_copy(dst_hbm.at[wid, 0], idst[s % 2],
                                  dsem[s % 2]).wait()
        gwait(k)
        scat(c, k)
        if c >= 1:
            scwait((c - 1) % DEPTH)
            if c % SUP == 0 and (c // SUP) + 1 < NSUP:
                pltpu.async_copy(dst_hbm.at[wid, s + 1], idst[(s + 1) % 2],
                                 dsem[(s + 1) % 2])
        g = c + DEPTH - 1
        if g < NCHUNK:
            sg = g // SUP
            if g % SUP == 0 and sg >= 1:
                pltpu.make_async_copy(src_hbm.at[wid, 0], isrc[sg % 2],
                                      ssem[sg % 2]).wait()
            gstart(g, (c - 1) % DEPTH)
        if c % SUP == SUP - 1 and s + 2 < NSUP:
            pltpu.async_copy(src_hbm.at[wid, s + 2], isrc[s % 2], ssem[s % 2])
    scwait((NCHUNK - 1) % DEPTH)

    plsc.subcore_barrier()
    pltpu.sync_copy(acc.at[pl.ds(sid * ROWS_PER_TILE, ROWS_PER_TILE)],
                    out_hbm.at[cid, pl.ds(sid * ROWS_PER_TILE, ROWS_PER_TILE)])


@jax.jit
def _sc_agg(src, dst, table, zeros_blk):
    mesh = plsc.VectorSubcoreMesh(core_axis_name="c", subcore_axis_name="s")
    return pl.kernel(
        _sc_agg_body,
        out_type=jax.ShapeDtypeStruct((2, NACC, D), jnp.float32),
        mesh=mesh,
        scratch_types=[
            pltpu.VMEM((SUP, CH), jnp.int32),
            pltpu.VMEM((SUP, CH), jnp.int32),
            pltpu.VMEM((SUP, CH), jnp.int32),
            pltpu.VMEM((SUP, CH), jnp.int32),
            pltpu.VMEM((CH, D), jnp.float32),
            pltpu.VMEM((CH, D), jnp.float32),
            pltpu.VMEM((CH, D), jnp.float32),
            pltpu.VMEM((CH, D), jnp.float32),
            pltpu.VMEM_SHARED((NACC, D), jnp.float32),
            pltpu.SemaphoreType.DMA,
            pltpu.SemaphoreType.DMA,
            pltpu.SemaphoreType.DMA,
            pltpu.SemaphoreType.DMA,
            pltpu.SemaphoreType.DMA,
            pltpu.SemaphoreType.DMA,
            pltpu.SemaphoreType.DMA,
            pltpu.SemaphoreType.DMA,
            pltpu.SemaphoreType.DMA,
            pltpu.SemaphoreType.DMA,
            pltpu.SemaphoreType.DMA,
            pltpu.SemaphoreType.DMA,
            pltpu.SemaphoreType.DMA,
        ],
    )(src, dst, table, zeros_blk)


def _mlp1_body(x_ref, agg_ref, w1_ref, b1_ref, w2_ref, b2_ref, eps_ref, o_ref):
    a = agg_ref[0, :N, :] + agg_ref[1, :N, :]
    xt = x_ref[...] * eps_ref[...] + a
    h1 = jnp.maximum(
        jnp.dot(xt, w1_ref[...], preferred_element_type=jnp.float32)
        + b1_ref[...], 0.0)
    h2 = jnp.dot(h1, w2_ref[...], preferred_element_type=jnp.float32) + b2_ref[...]
    o_ref[...] = jnp.maximum(h2, 0.0)


@jax.jit
def _mlp1(xp, aggs, W1, b1, W2, b2, epsv):
    return pl.pallas_call(
        _mlp1_body,
        out_shape=jax.ShapeDtypeStruct((N, D), jnp.float32),
    )(xp, aggs, W1, b1, W2, b2, epsv)


def _mlp2_pool_body(h_ref, agg_ref, w1_ref, b1_ref, w2_ref, b2_ref, eps_ref,
                    batch_ref, wl_ref, bl_ref, o_ref):
    a = agg_ref[0, :N, :] + agg_ref[1, :N, :]
    xt = h_ref[...] * eps_ref[...] + a
    h1 = jnp.maximum(
        jnp.dot(xt, w1_ref[...], preferred_element_type=jnp.float32)
        + b1_ref[...], 0.0)
    h2 = jnp.dot(h1, w2_ref[...], preferred_element_type=jnp.float32) + b2_ref[...]
    gids = lax.broadcasted_iota(jnp.int32, (G, N), 0)
    onehot = (batch_ref[...] == gids).astype(jnp.float32)
    sums = jnp.dot(onehot, h2, preferred_element_type=jnp.float32)
    cnt = jnp.sum(onehot, axis=1, keepdims=True)
    pooled = sums / jnp.maximum(cnt, 1.0)
    logits = jnp.dot(pooled, wl_ref[...], preferred_element_type=jnp.float32) \
        + bl_ref[...]
    m = jnp.max(logits, axis=-1, keepdims=True)
    lse = m + jnp.log(jnp.sum(jnp.exp(logits - m), axis=-1, keepdims=True))
    o_ref[...] = logits - lse


@jax.jit
def _mlp2_pool(h, aggs, W1, b1, W2, b2, epsv, batch_r, Wl, bl):
    return pl.pallas_call(
        _mlp2_pool_body,
        out_shape=jax.ShapeDtypeStruct((G, C), jnp.float32),
    )(h, aggs, W1, b1, W2, b2, epsv, batch_r, Wl, bl)


def kernel(x, edge_index, batch, eps1, W11, b11, W12, b12,
           eps2, W21, b21, W22, b22, Wl, bl):
    src = jnp.reshape(edge_index[0], (NWORKERS, NSUP, SUP, CH))
    dst = jnp.reshape(edge_index[1], (NWORKERS, NSUP, SUP, CH))
    batch_r = jnp.reshape(batch, (1, N))
    zeros_blk = jnp.zeros((ROWS_PER_TILE, D), jnp.float32)
    eps1v = jnp.broadcast_to(jnp.reshape(1.0 + eps1, (1, 1)), (1, D))
    eps2v = jnp.broadcast_to(jnp.reshape(1.0 + eps2, (1, 1)), (1, D))
    b11r = jnp.reshape(b11, (1, D))
    b12r = jnp.reshape(b12, (1, D))
    b21r = jnp.reshape(b21, (1, D))
    b22r = jnp.reshape(b22, (1, D))
    blr = jnp.reshape(bl, (1, C))

    aggs1 = _sc_agg(src, dst, x, zeros_blk)
    h = _mlp1(x, aggs1, W11, b11r, W12, b12r, eps1v)
    aggs2 = _sc_agg(src, dst, h, zeros_blk)
    return _mlp2_pool(h, aggs2, W21, b21r, W22, b22r, eps2v, batch_r, Wl, blr)


# speedup vs baseline: 1.1002x; 1.1002x over previous
import functools

import jax
import jax.numpy as jnp
from jax import lax
from jax.experimental import pallas as pl
from jax.experimental.pallas import tpu as pltpu
from jax.experimental.pallas import tpu_sc as plsc

N = 10000
E = 320000
D = 128
G = 32
C = 10
NACC = 10112
NWORKERS = 32
EPT = E // NWORKERS
CH = 80
NCHUNK = EPT // CH
DEPTH = 4
SUP = 5
NSUP = NCHUNK // SUP
ROWS_PER_TILE = NACC // 16


def _sc_agg_body(src_hbm, dst_hbm, table_hbm, zeros_hbm, out_hbm,
                 isrcA, idstA, isrcB, idstB, rows0, rows1, rows2, rows3, acc,
                 sem0, sem1, sem2, sem3, ssemA, ssemB, dsemA, dsemB, zsem):
    cid = lax.axis_index("c")
    sid = lax.axis_index("s")
    wid = cid * 16 + sid
    rows = [rows0, rows1, rows2, rows3]
    sems = [sem0, sem1, sem2, sem3]
    isrc = [isrcA, isrcB]
    idst = [idstA, idstB]
    ssem = [ssemA, ssemB]
    dsem = [dsemA, dsemB]

    pltpu.async_copy(zeros_hbm, rows0, zsem)
    pltpu.sync_copy(src_hbm.at[wid, 0], isrcA)
    pltpu.sync_copy(dst_hbm.at[wid, 0], idstA)
    pltpu.async_copy(src_hbm.at[wid, 1], isrcB, ssemB)
    pltpu.async_copy(dst_hbm.at[wid, 1], idstB, dsemB)
    pltpu.make_async_copy(zeros_hbm, rows0, zsem).wait()
    base = sid * ROWS_PER_TILE
    for t in range(ROWS_PER_TILE // CH):
        pltpu.sync_copy(rows0, acc.at[pl.ds(base + t * CH, CH)])
    rem = ROWS_PER_TILE % CH
    if rem:
        pltpu.sync_copy(rows0.at[pl.ds(0, rem)],
                        acc.at[pl.ds(base + ROWS_PER_TILE - rem, rem)])
    plsc.subcore_barrier()

    def gstart(c, k):
        p, j = (c // SUP) % 2, c % SUP
        pltpu.async_copy(table_hbm.at[isrc[p].at[j]], rows[k], sems[k])

    def gwait(k):
        pltpu.make_async_copy(table_hbm.at[isrcA.at[0]], rows[k], sems[k]).wait()

    def scat(c, k):
        p, j = (c // SUP) % 2, c % SUP
        pltpu.sync_copy(rows[k], acc.at[idst[p].at[j]], add=True)

    for c in range(DEPTH):
        gstart(c, c)
    for c in range(NCHUNK):
        k = c % DEPTH
        s = c // SUP
        if c % SUP == 0 and s >= 1:
            pltpu.make_async_copy(dst_hbm.at[wid, 0], idst[s % 2],
                                  dsem[s % 2]).wait()
        gwait(k)
        scat(c, k)
        g = c + DEPTH
        if g < NCHUNK:
            sg = g // SUP
            if g % SUP == 0 and sg >= 1:
                pltpu.make_async_copy(src_hbm.at[wid, 0], isrc[sg % 2],
                                      ssem[sg % 2]).wait()
            gstart(g, k)
        if c % SUP == SUP - 1 and s + 2 < NSUP:
            pltpu.async_copy(src_hbm.at[wid, s + 2], isrc[s % 2], ssem[s % 2])
            pltpu.async_copy(dst_hbm.at[wid, s + 2], idst[s % 2], dsem[s % 2])

    plsc.subcore_barrier()
    pltpu.sync_copy(acc.at[pl.ds(sid * ROWS_PER_TILE, ROWS_PER_TILE)],
                    out_hbm.at[cid, pl.ds(sid * ROWS_PER_TILE, ROWS_PER_TILE)])


@jax.jit
def _sc_agg(src, dst, table, zeros_blk):
    mesh = plsc.VectorSubcoreMesh(core_axis_name="c", subcore_axis_name="s")
    return pl.kernel(
        _sc_agg_body,
        out_type=jax.ShapeDtypeStruct((2, NACC, D), jnp.float32),
        mesh=mesh,
        scratch_types=[
            pltpu.VMEM((SUP, CH), jnp.int32),
            pltpu.VMEM((SUP, CH), jnp.int32),
            pltpu.VMEM((SUP, CH), jnp.int32),
            pltpu.VMEM((SUP, CH), jnp.int32),
            pltpu.VMEM((CH, D), jnp.float32),
            pltpu.VMEM((CH, D), jnp.float32),
            pltpu.VMEM((CH, D), jnp.float32),
            pltpu.VMEM((CH, D), jnp.float32),
            pltpu.VMEM_SHARED((NACC, D), jnp.float32),
            pltpu.SemaphoreType.DMA,
            pltpu.SemaphoreType.DMA,
            pltpu.SemaphoreType.DMA,
            pltpu.SemaphoreType.DMA,
            pltpu.SemaphoreType.DMA,
            pltpu.SemaphoreType.DMA,
            pltpu.SemaphoreType.DMA,
            pltpu.SemaphoreType.DMA,
            pltpu.SemaphoreType.DMA,
        ],
    )(src, dst, table, zeros_blk)


def _mlp1_body(x_ref, agg_ref, w1_ref, b1_ref, w2_ref, b2_ref, eps_ref, o_ref):
    a = agg_ref[0, :N, :] + agg_ref[1, :N, :]
    xt = x_ref[...] * eps_ref[...] + a
    h1 = jnp.maximum(
        jnp.dot(xt, w1_ref[...], preferred_element_type=jnp.float32)
        + b1_ref[...], 0.0)
    h2 = jnp.dot(h1, w2_ref[...], preferred_element_type=jnp.float32) + b2_ref[...]
    o_ref[...] = jnp.maximum(h2, 0.0)


@jax.jit
def _mlp1(xp, aggs, W1, b1, W2, b2, epsv):
    return pl.pallas_call(
        _mlp1_body,
        out_shape=jax.ShapeDtypeStruct((N, D), jnp.float32),
    )(xp, aggs, W1, b1, W2, b2, epsv)


def _mlp2_pool_body(h_ref, agg_ref, w1_ref, b1_ref, w2_ref, b2_ref, eps_ref,
                    batch_ref, wl_ref, bl_ref, o_ref):
    a = agg_ref[0, :N, :] + agg_ref[1, :N, :]
    xt = h_ref[...] * eps_ref[...] + a
    h1 = jnp.maximum(
        jnp.dot(xt, w1_ref[...], preferred_element_type=jnp.float32)
        + b1_ref[...], 0.0)
    h2 = jnp.dot(h1, w2_ref[...], preferred_element_type=jnp.float32) + b2_ref[...]
    gids = lax.broadcasted_iota(jnp.int32, (G, N), 0)
    onehot = (batch_ref[...] == gids).astype(jnp.float32)
    sums = jnp.dot(onehot, h2, preferred_element_type=jnp.float32)
    cnt = jnp.sum(onehot, axis=1, keepdims=True)
    pooled = sums / jnp.maximum(cnt, 1.0)
    logits = jnp.dot(pooled, wl_ref[...], preferred_element_type=jnp.float32) \
        + bl_ref[...]
    m = jnp.max(logits, axis=-1, keepdims=True)
    lse = m + jnp.log(jnp.sum(jnp.exp(logits - m), axis=-1, keepdims=True))
    o_ref[...] = logits - lse


@jax.jit
def _mlp2_pool(h, aggs, W1, b1, W2, b2, epsv, batch_r, Wl, bl):
    return pl.pallas_call(
        _mlp2_pool_body,
        out_shape=jax.ShapeDtypeStruct((G, C), jnp.float32),
    )(h, aggs, W1, b1, W2, b2, epsv, batch_r, Wl, bl)


def kernel(x, edge_index, batch, eps1, W11, b11, W12, b12,
           eps2, W21, b21, W22, b22, Wl, bl):
    src = jnp.reshape(edge_index[0], (NWORKERS, NSUP, SUP, CH))
    dst = jnp.reshape(edge_index[1], (NWORKERS, NSUP, SUP, CH))
    batch_r = jnp.reshape(batch, (1, N))
    zeros_blk = jnp.zeros((CH, D), jnp.float32)
    eps1v = jnp.broadcast_to(jnp.reshape(1.0 + eps1, (1, 1)), (1, D))
    eps2v = jnp.broadcast_to(jnp.reshape(1.0 + eps2, (1, 1)), (1, D))
    b11r = jnp.reshape(b11, (1, D))
    b12r = jnp.reshape(b12, (1, D))
    b21r = jnp.reshape(b21, (1, D))
    b22r = jnp.reshape(b22, (1, D))
    blr = jnp.reshape(bl, (1, C))

    aggs1 = _sc_agg(src, dst, x, zeros_blk)
    h = _mlp1(x, aggs1, W11, b11r, W12, b12r, eps1v)
    aggs2 = _sc_agg(src, dst, h, zeros_blk)
    return _mlp2_pool(h, aggs2, W21, b21r, W22, b22r, eps2v, batch_r, Wl, blr)
